# baseline (device time: 9859 ns/iter reference)
import jax
import jax.numpy as jnp
from jax import lax
from jax.experimental import pallas as pl
from jax.experimental.pallas import tpu as pltpu

K = 8
NEG_INF = float("-inf")


def _topk_desc(vals, k):
    cols = []
    for _ in range(k):
        mk = jnp.max(vals, axis=1, keepdims=True)
        cols.append(mk)
        vals = jnp.where(vals == mk, NEG_INF, vals)
    return jnp.concatenate(cols, axis=1)


def kernel(x):
    m, n = x.shape
    half = n // 2

    def body(x_ref, out_ref, mine_ref, theirs_ref, send_sems, recv_sems):
        my_x = lax.axis_index("x")
        my_y = lax.axis_index("y")
        my_z = lax.axis_index("z")
        partner = (1 - my_x, my_y, my_z)

        barrier_sem = pltpu.get_barrier_semaphore()
        pl.semaphore_signal(
            barrier_sem, inc=1,
            device_id=partner, device_id_type=pl.DeviceIdType.MESH,
        )

        mine_ref[0] = _topk_desc(x_ref[:, :half], K)
        pl.semaphore_wait(barrier_sem, 1)
        rdma0 = pltpu.make_async_remote_copy(
            src_ref=mine_ref.at[0],
            dst_ref=theirs_ref.at[0],
            send_sem=send_sems.at[0],
            recv_sem=recv_sems.at[0],
            device_id=partner,
            device_id_type=pl.DeviceIdType.MESH,
        )
        rdma0.start()

        mine_ref[1] = _topk_desc(x_ref[:, half:], K)
        rdma1 = pltpu.make_async_remote_copy(
            src_ref=mine_ref.at[1],
            dst_ref=theirs_ref.at[1],
            send_sem=send_sems.at[1],
            recv_sem=recv_sems.at[1],
            device_id=partner,
            device_id_type=pl.DeviceIdType.MESH,
        )
        rdma1.start()

        rdma0.wait()
        rdma1.wait()

        cand = jnp.concatenate(
            [mine_ref[0], mine_ref[1], theirs_ref[0], theirs_ref[1]], axis=1
        )
        out_ref[:, :] = _topk_desc(cand, K)

    return pl.pallas_call(
        body,
        out_shape=jax.ShapeDtypeStruct((m, K), jnp.float32),
        in_specs=[pl.BlockSpec(memory_space=pltpu.VMEM)],
        out_specs=pl.BlockSpec(memory_space=pltpu.VMEM),
        scratch_shapes=[
            pltpu.VMEM((2, m, K), jnp.float32),
            pltpu.VMEM((2, m, K), jnp.float32),
            pltpu.SemaphoreType.DMA((2,)),
            pltpu.SemaphoreType.DMA((2,)),
        ],
        compiler_params=pltpu.CompilerParams(collective_id=0),
    )(x)


# device time: 8491 ns/iter; 1.1611x vs baseline; 1.1611x over previous
import jax
import jax.numpy as jnp
from jax import lax
from jax.experimental import pallas as pl
from jax.experimental.pallas import tpu as pltpu

K = 8
NEG_INF = float("-inf")


def _topk_desc(vals, k):
    cols = []
    for _ in range(k):
        mk = jnp.max(vals, axis=1, keepdims=True)
        cols.append(mk)
        vals = jnp.where(vals == mk, NEG_INF, vals)
    return jnp.concatenate(cols, axis=1)


def _local_top8(x):
    n_chunks = K
    chunk = x.shape[1] // n_chunks
    v = [x[:, i * chunk : (i + 1) * chunk] for i in range(n_chunks)]

    for r in range(n_chunks):
        for i in range(r % 2, n_chunks - 1, 2):
            hi = jnp.maximum(v[i], v[i + 1])
            lo = jnp.minimum(v[i], v[i + 1])
            v[i], v[i + 1] = hi, lo

    cols = []
    for k in range(K):
        mk = jnp.max(v[0], axis=1, keepdims=True)
        cols.append(mk)
        depth = K - k
        if depth > 1:
            mask = v[0] == mk
            for j in range(depth - 1):
                v[j] = jnp.where(mask, v[j + 1], v[j])
            v[depth - 1] = jnp.where(mask, NEG_INF, v[depth - 1])
    return jnp.concatenate(cols, axis=1)


def kernel(x):
    m, n = x.shape

    def body(x_ref, out_ref, mine_ref, theirs_ref, send_sem, recv_sem):
        my_x = lax.axis_index("x")
        my_y = lax.axis_index("y")
        my_z = lax.axis_index("z")
        partner = (1 - my_x, my_y, my_z)

        barrier_sem = pltpu.get_barrier_semaphore()
        pl.semaphore_signal(
            barrier_sem, inc=1,
            device_id=partner, device_id_type=pl.DeviceIdType.MESH,
        )

        mine_ref[:, :] = _local_top8(x_ref[:, :])

        pl.semaphore_wait(barrier_sem, 1)

        rdma = pltpu.make_async_remote_copy(
            src_ref=mine_ref,
            dst_ref=theirs_ref,
            send_sem=send_sem,
            recv_sem=recv_sem,
            device_id=partner,
            device_id_type=pl.DeviceIdType.MESH,
        )
        rdma.start()
        rdma.wait()

        cand = jnp.concatenate([mine_ref[:, :], theirs_ref[:, :]], axis=1)
        out_ref[:, :] = _topk_desc(cand, K)

    return pl.pallas_call(
        body,
        out_shape=jax.ShapeDtypeStruct((m, K), jnp.float32),
        in_specs=[pl.BlockSpec(memory_space=pltpu.VMEM)],
        out_specs=pl.BlockSpec(memory_space=pltpu.VMEM),
        scratch_shapes=[
            pltpu.VMEM((m, K), jnp.float32),
            pltpu.VMEM((m, K), jnp.float32),
            pltpu.SemaphoreType.DMA,
            pltpu.SemaphoreType.DMA,
        ],
        compiler_params=pltpu.CompilerParams(collective_id=0),
    )(x)
